# Initial kernel scaffold; baseline (speedup 1.0000x reference)
import jax
import jax.numpy as jnp
from jax.experimental import pallas as pl


def kernel(type, x_num, edge_index, batch, W1, b1, W2, b2, Wc, bc, gc, betac, Wn, bn, gn, betan, Wf, bf, Wo, bo):
    raise NotImplementedError("write your pallas kernel here")


# speedup vs baseline: 4.5264x; 4.5264x over previous
import functools

import jax
import jax.numpy as jnp
from jax import lax
from jax.experimental import pallas as pl
from jax.experimental.pallas import tpu as pltpu
from jax.experimental.pallas import tpu_sc as plsc

N = 10000
E = 640000
H = 128
G = 64
TILES = 16
W = 2 * TILES
NPAD = 10240
NBLK = 10
BLK = NPAD // NBLK
EB = 128
E2 = 655360
RPT = NPAD // TILES


@functools.cache
def _mesh():
    return plsc.VectorSubcoreMesh(core_axis_name="c", subcore_axis_name="s",
                                  num_cores=2)


_SC_PARAMS = pltpu.CompilerParams(needs_layout_passes=False)


def _agg_pass(h_hbm, out_hbm, src_hbm, dst_hbm, idxs_v, idxd_v, rows_v, sem,
              acc_sh, sid):
    ept = E2 // TILES
    pltpu.sync_copy(h_hbm.at[pl.ds(sid * RPT, RPT)],
                    acc_sh.at[pl.ds(sid * RPT, RPT)])
    plsc.subcore_barrier()

    def chunk(j, c):
        base = sid * ept + j * EB
        pltpu.sync_copy(src_hbm.at[pl.ds(base, EB)], idxs_v)
        pltpu.sync_copy(dst_hbm.at[pl.ds(base, EB)], idxd_v)
        pltpu.async_copy(h_hbm.at[idxs_v], rows_v, sem).wait()
        pltpu.sync_copy(rows_v, acc_sh.at[idxd_v], add=True)
        return c
    lax.fori_loop(0, ept // EB, chunk, 0)
    plsc.subcore_barrier()
    pltpu.sync_copy(acc_sh.at[pl.ds(sid * RPT, RPT)],
                    out_hbm.at[pl.ds(sid * RPT, RPT)])


def _agg_body(hc_hbm, hn_hbm, src_hbm, dst_hbm, outc_hbm, outn_hbm,
              idxs_v, idxd_v, rows_v, sem, acc_sh):
    cid = lax.axis_index("c")
    sid = lax.axis_index("s")

    @pl.when(cid == 0)
    def _():
        _agg_pass(hc_hbm, outc_hbm, src_hbm, dst_hbm, idxs_v, idxd_v,
                  rows_v, sem, acc_sh, sid)

    @pl.when(cid == 1)
    def _():
        _agg_pass(hn_hbm, outn_hbm, src_hbm, dst_hbm, idxs_v, idxd_v,
                  rows_v, sem, acc_sh, sid)


@functools.cache
def _sc_aggregate():
    return pl.kernel(
        _agg_body,
        out_type=[jax.ShapeDtypeStruct((NPAD, H), jnp.float32),
                  jax.ShapeDtypeStruct((NPAD, H), jnp.float32)],
        mesh=_mesh(),
        compiler_params=_SC_PARAMS,
        scratch_types=[
            pltpu.VMEM((EB,), jnp.int32),
            pltpu.VMEM((EB,), jnp.int32),
            pltpu.VMEM((EB, H), jnp.float32),
            pltpu.SemaphoreType.DMA,
            pltpu.VMEM_SHARED((NPAD, H), jnp.float32),
        ],
    )


def _elu(x):
    return jnp.where(x > 0.0, x, jnp.exp(jnp.minimum(x, 0.0)) - 1.0)


def _mm(a, b):
    return jnp.dot(a, b, preferred_element_type=jnp.float32)


def _ln(y, g, b):
    mu = jnp.mean(y, axis=-1, keepdims=True)
    var = jnp.mean((y - mu) ** 2, axis=-1, keepdims=True)
    return (y - mu) * lax.rsqrt(var + 1e-5) * g + b


def _prep_body(typef_ref, xn_ref, degp_ref, w1_ref, b1_ref, w2_ref, b2_ref,
               wc0_ref, wn0_ref, hc_ref, hn_ref, dinv_ref):
    t = typef_ref[...]
    iota = lax.broadcasted_iota(jnp.int32, (BLK, H), 1).astype(jnp.float32)
    onehot = (iota == t).astype(jnp.float32)
    xt = _mm(onehot, w1_ref[...]) + b1_ref[...]
    xn = _mm(xn_ref[...], w2_ref[...]) + b2_ref[...]
    deg = degp_ref[...][:, :1]
    dinv = lax.rsqrt(deg)
    dinv_ref[...] = jnp.broadcast_to(dinv, (BLK, 8))
    hc_ref[...] = _mm(xt, wc0_ref[...]) * dinv
    hn_ref[...] = _mm(xn, wn0_ref[...]) * dinv


def _mid_body(aggc_ref, aggn_ref, dinv_ref, bc_ref, gc_ref, betac_ref,
              bn_ref, gn_ref, betan_ref, wc_ref, wn_ref, hc_ref, hn_ref):
    dinv = dinv_ref[...][:, :1]

    def side(agg, b, g, beta, w):
        y = _elu(_ln(dinv * agg + b, g, beta))
        return _mm(y, w) * dinv

    hc_ref[...] = side(aggc_ref[...], bc_ref[...], gc_ref[...],
                       betac_ref[...], wc_ref[...])
    hn_ref[...] = side(aggn_ref[...], bn_ref[...], gn_ref[...],
                       betan_ref[...], wn_ref[...])


def _final_body(aggc_ref, aggn_ref, dinv_ref, bc_ref, bn_ref, batchf_ref,
                wf1_ref, wf2_ref, bf_ref, wo_ref, bo_ref,
                out_ref, pooled_ref, cnt_ref):
    i = pl.program_id(0)
    dinv = dinv_ref[...][:, :1]
    zc = _elu(dinv * aggc_ref[...] + bc_ref[...])
    zn = _elu(dinv * aggn_ref[...] + bn_ref[...])
    z = _elu(_mm(zc, wf1_ref[...]) + _mm(zn, wf2_ref[...]) + bf_ref[...])
    bt = batchf_ref[...]
    iota = lax.broadcasted_iota(jnp.int32, (BLK, G), 1).astype(jnp.float32)
    mask = (iota == bt).astype(jnp.float32)
    contrib = lax.dot_general(mask, z, (((0,), (0,)), ((), ())),
                              preferred_element_type=jnp.float32)
    cnt_c = lax.dot_general(mask, jnp.ones((BLK, 8), jnp.float32),
                            (((0,), (0,)), ((), ())),
                            preferred_element_type=jnp.float32)

    @pl.when(i == 0)
    def _():
        pooled_ref[...] = contrib
        cnt_ref[...] = cnt_c

    @pl.when(i > 0)
    def _():
        pooled_ref[...] = pooled_ref[...] + contrib
        cnt_ref[...] = cnt_ref[...] + cnt_c

    pooled = pooled_ref[...] / jnp.maximum(cnt_ref[...][:, :1], 1.0)
    out_ref[...] = _mm(pooled, wo_ref[...]) + bo_ref[...]


def _rowspec():
    return pl.BlockSpec((BLK, H), lambda i: (i, 0))


def _fullspec(shape):
    return pl.BlockSpec(shape, lambda i: tuple(0 for _ in shape))


def kernel(type, x_num, edge_index, batch, W1, b1, W2, b2, Wc, bc, gc, betac,
           Wn, bn, gn, betan, Wf, bf, Wo, bo):
    f32 = jnp.float32
    pad_e = jnp.full((E2 - E,), NPAD - 1, jnp.int32)
    src = jnp.concatenate([edge_index[0].astype(jnp.int32), pad_e])
    dst = jnp.concatenate([edge_index[1].astype(jnp.int32), pad_e])
    typef = jnp.pad(type.astype(f32), (0, NPAD - N),
                    constant_values=float(H)).reshape(NPAD, 1)
    batchf = jnp.pad(batch.astype(f32), (0, NPAD - N),
                     constant_values=float(G)).reshape(NPAD, 1)
    xnp = jnp.pad(x_num, ((0, NPAD - N), (0, 8 - x_num.shape[1])))
    w1p = jnp.pad(W1, ((0, H - W1.shape[0]), (0, 0)))
    w2p = jnp.pad(W2, ((0, 8 - W2.shape[0]), (0, 0)))
    wop = jnp.pad(Wo, ((0, 0), (0, H - Wo.shape[1])))
    bop = jnp.pad(bo, (0, H - bo.shape[0])).reshape(1, H)
    row = lambda v: v.reshape(1, H)

    onesf = jnp.ones((NPAD, H), f32)
    degp, _ = _sc_aggregate()(onesf, onesf, src, dst)

    nmshape = jax.ShapeDtypeStruct((NPAD, H), f32)
    prep = pl.pallas_call(
        _prep_body,
        grid=(NBLK,),
        in_specs=[
            pl.BlockSpec((BLK, 1), lambda i: (i, 0)),
            pl.BlockSpec((BLK, 8), lambda i: (i, 0)),
            _rowspec(),
            _fullspec((H, H)), _fullspec((1, H)),
            _fullspec((8, H)), _fullspec((1, H)),
            _fullspec((H, H)), _fullspec((H, H)),
        ],
        out_specs=[_rowspec(), _rowspec(),
                   pl.BlockSpec((BLK, 8), lambda i: (i, 0))],
        out_shape=[nmshape, nmshape, jax.ShapeDtypeStruct((NPAD, 8), f32)],
    )
    hcb, hnb, dinv2 = prep(typef, xnp, degp, w1p, row(b1), w2p, row(b2),
                           Wc[0], Wn[0])

    mid = pl.pallas_call(
        _mid_body,
        grid=(NBLK,),
        in_specs=[_rowspec(), _rowspec(),
                  pl.BlockSpec((BLK, 8), lambda i: (i, 0))]
                 + [_fullspec((1, H))] * 6
                 + [_fullspec((H, H)), _fullspec((H, H))],
        out_specs=[_rowspec(), _rowspec()],
        out_shape=[nmshape, nmshape],
    )

    for i in range(2):
        aggc, aggn = _sc_aggregate()(hcb, hnb, src, dst)
        hcb, hnb = mid(aggc, aggn, dinv2,
                       row(bc[i]), row(gc[i]), row(betac[i]),
                       row(bn[i]), row(gn[i]), row(betan[i]),
                       Wc[i + 1], Wn[i + 1])

    aggc, aggn = _sc_aggregate()(hcb, hnb, src, dst)

    final = pl.pallas_call(
        _final_body,
        grid=(NBLK,),
        in_specs=[_rowspec(), _rowspec(),
                  pl.BlockSpec((BLK, 8), lambda i: (i, 0)),
                  _fullspec((1, H)), _fullspec((1, H)),
                  pl.BlockSpec((BLK, 1), lambda i: (i, 0)),
                  _fullspec((H, H)), _fullspec((H, H)), _fullspec((1, H)),
                  _fullspec((H, H)), _fullspec((1, H))],
        out_specs=[_fullspec((G, H)), _fullspec((G, H)), _fullspec((G, 8))],
        out_shape=[jax.ShapeDtypeStruct((G, H), f32),
                   jax.ShapeDtypeStruct((G, H), f32),
                   jax.ShapeDtypeStruct((G, 8), f32)],
    )
    outp, _, _ = final(aggc, aggn, dinv2, row(bc[2]), row(bn[2]), batchf,
                       Wf[:H], Wf[H:], row(bf), wop, bop)
    return outp[:, :4]
